# Initial kernel scaffold; baseline (speedup 1.0000x reference)
import jax
import jax.numpy as jnp
from jax.experimental import pallas as pl


def kernel(x, params):
    raise NotImplementedError("write your pallas kernel here")


# speedup vs baseline: 11.0344x; 11.0344x over previous
import functools

import jax
import jax.numpy as jnp
from jax import lax
from jax.experimental import pallas as pl
from jax.experimental.pallas import tpu as pltpu
from jax.experimental.pallas import tpu_sc as plsc

V = 32000
D = 768
SS = 16
E = 4
DFF = 4 * D
L = 2048

T_CHUNK = 128
M_BLK = 256
F_BLK = 512
N_BLK = 3200


def _embed_gather(table, idx):
    info = plsc.get_sparse_core_info()
    nw = info.num_cores * info.num_subcores
    b_per_w = L // nw
    mesh = plsc.VectorSubcoreMesh(core_axis_name="c", subcore_axis_name="s")

    @functools.partial(
        pl.kernel,
        mesh=mesh,
        out_type=jax.ShapeDtypeStruct((L, D), jnp.float32),
        scratch_types=[
            pltpu.VMEM((b_per_w,), jnp.int32),
            pltpu.VMEM((b_per_w, D), jnp.float32),
            pltpu.SemaphoreType.DMA,
        ],
    )
    def k(table_hbm, idx_hbm, out_hbm, idx_v, rows_v, sem):
        wid = lax.axis_index("s") * info.num_cores + lax.axis_index("c")
        base = wid * b_per_w
        pltpu.sync_copy(idx_hbm.at[pl.ds(base, b_per_w)], idx_v)
        pltpu.async_copy(table_hbm.at[idx_v], rows_v, sem).wait()
        pltpu.sync_copy(rows_v, out_hbm.at[pl.ds(base, b_per_w)])

    return k(table, idx)


def _scan_body(x_ref, dw_ref, db_ref, w2t_ref, bcb_ref, at_ref, dp_ref,
               y_ref, dstate, h_s):
    gi = pl.program_id(0)

    @pl.when(gi == 0)
    def _():
        h_s[...] = jnp.zeros_like(h_s)

    xb = x_ref[...]
    delta = jnp.dot(xb, dw_ref[...], preferred_element_type=jnp.float32)
    delta = delta + db_ref[...]
    delta = jnp.log(1.0 + jnp.exp(-jnp.abs(delta))) + jnp.maximum(delta, 0.0)
    dstate[...] = delta
    bct = lax.dot_general(w2t_ref[...], xb, (((1,), (1,)), ((), ())),
                          preferred_element_type=jnp.float32)
    bct = bct + bcb_ref[...]

    at = at_ref[...]
    lane_iota = lax.broadcasted_iota(jnp.int32, (1, T_CHUNK), 1)

    def step(t, h):
        oh = (lane_iota == t).astype(jnp.float32)
        bc_col = jnp.sum(bct * oh, axis=1, keepdims=True)
        b_col = bc_col[0:SS, :]
        c_col = bc_col[SS:2 * SS, :]
        d_t = dstate[pl.ds(t, 1), :]
        x_t = x_ref[pl.ds(t, 1), :]
        a = jnp.exp(jnp.minimum(d_t * at, 2.0))
        bb = jnp.clip(d_t * b_col, -2.0, 2.0)
        h = a * h + bb * x_t
        h = jnp.clip(h, -100.0, 100.0)
        y = jnp.sum(h * c_col, axis=0, keepdims=True)
        y_ref[pl.ds(t, 1), :] = y
        return h

    h = lax.fori_loop(0, T_CHUNK, step, h_s[...])
    h_s[...] = h
    y_ref[...] = y_ref[...] + xb * dp_ref[...]


def _ssm_scan(x, dw, db, w2t, bcb, at, dp):
    grid = (L // T_CHUNK,)
    return pl.pallas_call(
        _scan_body,
        grid=grid,
        in_specs=[
            pl.BlockSpec((T_CHUNK, D), lambda i: (i, 0)),
            pl.BlockSpec((D, D), lambda i: (0, 0)),
            pl.BlockSpec((1, D), lambda i: (0, 0)),
            pl.BlockSpec((2 * SS, D), lambda i: (0, 0)),
            pl.BlockSpec((2 * SS, 1), lambda i: (0, 0)),
            pl.BlockSpec((SS, D), lambda i: (0, 0)),
            pl.BlockSpec((1, D), lambda i: (0, 0)),
        ],
        out_specs=pl.BlockSpec((T_CHUNK, D), lambda i: (i, 0)),
        out_shape=jax.ShapeDtypeStruct((L, D), jnp.float32),
        scratch_shapes=[
            pltpu.VMEM((T_CHUNK, D), jnp.float32),
            pltpu.VMEM((SS, D), jnp.float32),
        ],
    )(x, dw, db, w2t, bcb, at, dp)


def _moe_body(y_ref, rw_ref, rb_ref, up_ref, ub_ref, dwn_ref, dbn_ref,
              g_ref, b_ref, out_ref, acc_s, tw_s, ti_s):
    e = pl.program_id(0)
    f = pl.program_id(1)
    m = pl.program_id(2)
    n_e = pl.num_programs(0)
    n_f = pl.num_programs(1)
    rows = pl.ds(m * M_BLK, M_BLK)

    yb = y_ref[...]

    @pl.when((e == 0) & (f == 0))
    def _():
        logits = jnp.dot(yb, rw_ref[...], preferred_element_type=jnp.float32)
        logits = (logits + rb_ref[...])[:, 0:E]
        mx = jnp.max(logits, axis=1, keepdims=True)
        ex = jnp.exp(logits - mx)
        sm = ex / jnp.sum(ex, axis=1, keepdims=True)
        tw = jnp.max(sm, axis=1, keepdims=True)
        iot = lax.broadcasted_iota(jnp.int32, (M_BLK, E), 1)
        ti = jnp.min(jnp.where(sm >= tw, iot, E), axis=1, keepdims=True)
        tw_s[rows, :] = tw
        ti_s[rows, :] = ti

    hid = jnp.dot(yb, up_ref[0], preferred_element_type=jnp.float32)
    hid = hid + ub_ref[0]
    hid = hid / (1.0 + jnp.exp(-hid))
    part = jnp.dot(hid, dwn_ref[0], preferred_element_type=jnp.float32)

    gate = jnp.where(ti_s[rows, :] == e, tw_s[rows, :], 0.0)
    part = part + jnp.where(f == 0, 1.0, 0.0) * dbn_ref[0]
    contrib = gate * part

    @pl.when((e == 0) & (f == 0))
    def _():
        acc_s[rows, :] = contrib

    @pl.when(~((e == 0) & (f == 0)))
    def _():
        acc_s[rows, :] = acc_s[rows, :] + contrib

    @pl.when((e == n_e - 1) & (f == n_f - 1))
    def _():
        o = yb + acc_s[rows, :]
        mu = jnp.mean(o, axis=1, keepdims=True)
        oc = o - mu
        var = jnp.mean(oc * oc, axis=1, keepdims=True)
        out_ref[...] = oc * lax.rsqrt(var + 1e-5) * g_ref[...] + b_ref[...]


def _moe_ln(y, rw_p, rb_p, up_w, ub3, down_w, db3, ln_g, ln_b):
    grid = (E, DFF // F_BLK, L // M_BLK)
    return pl.pallas_call(
        _moe_body,
        grid=grid,
        in_specs=[
            pl.BlockSpec((M_BLK, D), lambda e, f, m: (m, 0)),
            pl.BlockSpec((D, 128), lambda e, f, m: (0, 0)),
            pl.BlockSpec((1, 128), lambda e, f, m: (0, 0)),
            pl.BlockSpec((1, D, F_BLK), lambda e, f, m: (e, 0, f)),
            pl.BlockSpec((1, 1, F_BLK), lambda e, f, m: (e * (DFF // F_BLK) + f, 0, 0)),
            pl.BlockSpec((1, F_BLK, D), lambda e, f, m: (e, f, 0)),
            pl.BlockSpec((1, 1, D), lambda e, f, m: (e, 0, 0)),
            pl.BlockSpec((1, D), lambda e, f, m: (0, 0)),
            pl.BlockSpec((1, D), lambda e, f, m: (0, 0)),
        ],
        out_specs=pl.BlockSpec((M_BLK, D), lambda e, f, m: (m, 0)),
        out_shape=jax.ShapeDtypeStruct((L, D), jnp.float32),
        scratch_shapes=[
            pltpu.VMEM((L, D), jnp.float32),
            pltpu.VMEM((L, 1), jnp.float32),
            pltpu.VMEM((L, 1), jnp.int32),
        ],
    )(y, rw_p, rb_p, up_w, ub3, down_w, db3, ln_g, ln_b)


def _unembed_body(x_ref, emb_ref, out_ref):
    out_ref[...] = lax.dot_general(
        x_ref[...], emb_ref[...], (((1,), (1,)), ((), ())),
        preferred_element_type=jnp.float32)


def _unembed(h, embed):
    grid = (V // N_BLK, L // M_BLK)
    return pl.pallas_call(
        _unembed_body,
        grid=grid,
        in_specs=[
            pl.BlockSpec((M_BLK, D), lambda n, m: (m, 0)),
            pl.BlockSpec((N_BLK, D), lambda n, m: (n, 0)),
        ],
        out_specs=pl.BlockSpec((M_BLK, N_BLK), lambda n, m: (m, n)),
        out_shape=jax.ShapeDtypeStruct((L, V), jnp.float32),
    )(h, embed)


def kernel(x, params):
    embed = params['embed']
    idx = x.reshape(-1).astype(jnp.int32)
    h = _embed_gather(embed, idx)
    for lp in params['layers']:
        at = (-jnp.exp(lp['A_log'])).T
        w2t = jnp.concatenate([lp['B_w'], lp['C_w']], axis=1).T
        bcb = jnp.concatenate([lp['B_b'], lp['C_b']])[:, None]
        y = _ssm_scan(h, lp['delta_w'], lp['delta_b'][None], w2t, bcb, at,
                      lp['Dp'][None])
        rw_p = jnp.pad(lp['router_w'], ((0, 0), (0, 128 - E)))
        rb_p = jnp.pad(lp['router_b'], (0, 128 - E))[None]
        ub3 = lp['up_b'].reshape(E * (DFF // F_BLK), 1, F_BLK)
        db3 = lp['down_b'][:, None, :]
        h = _moe_ln(y, rw_p, rb_p, lp['up_w'], ub3, lp['down_w'], db3,
                    lp['ln_g'][None], lp['ln_b'][None])
    logits = _unembed(h, embed)
    return logits[None]
